# Initial kernel scaffold; baseline (speedup 1.0000x reference)
import functools

import jax
import jax.numpy as jnp
from jax import lax
from jax.experimental import pallas as pl
from jax.experimental.pallas import tpu as pltpu
from jax.experimental.pallas import tpu_sc as plsc

EMBED = 32
BATCH = 16384
SEQ = 50

NC = 2
NS = 16
NW = NC * NS
ROWS_PER_W = BATCH // NW
CHUNK = 64
N_CHUNKS = ROWS_PER_W // CHUNK
IDX_PER_CHUNK = CHUNK * SEQ
GATHER = 128
N_GATHERS = IDX_PER_CHUNK // GATHER
IDX_ROWS_PER_CHUNK = IDX_PER_CHUNK // GATHER
INV_S = 1.0 / SEQ

_mesh = plsc.VectorSubcoreMesh(core_axis_name="c", subcore_axis_name="s")


@functools.partial(
    pl.kernel,
    mesh=_mesh,
    out_type=jax.ShapeDtypeStruct((BATCH, EMBED), jnp.float32),
    scratch_types=[
        pltpu.VMEM((IDX_ROWS_PER_CHUNK, GATHER), jnp.int32),
        pltpu.VMEM((IDX_PER_CHUNK, EMBED), jnp.float32),
        pltpu.VMEM((CHUNK, EMBED), jnp.float32),
        pltpu.SemaphoreType.DMA,
    ],
)
def _emb(idx_hbm, table_hbm, out_hbm, idx_v, rows_v, out_v, sem):
    wid = lax.axis_index("s") * NC + lax.axis_index("c")

    def chunk_body(k, carry):
        chunk_id = wid * N_CHUNKS + k
        pltpu.sync_copy(
            idx_hbm.at[pl.ds(chunk_id * IDX_ROWS_PER_CHUNK, IDX_ROWS_PER_CHUNK)],
            idx_v,
        )
        copies = [
            pltpu.async_copy(
                table_hbm.at[idx_v.at[j]],
                rows_v.at[pl.ds(j * GATHER, GATHER)],
                sem,
            )
            for j in range(N_GATHERS)
        ]
        for c in copies:
            c.wait()

        def row_body(c, carry2):
            base = c * SEQ
            a0 = rows_v[base, pl.ds(0, 16)]
            a1 = rows_v[base, pl.ds(16, 16)]
            b0 = rows_v[base + 1, pl.ds(0, 16)]
            b1 = rows_v[base + 1, pl.ds(16, 16)]
            for s in range(2, SEQ, 2):
                a0 = a0 + rows_v[base + s, pl.ds(0, 16)]
                a1 = a1 + rows_v[base + s, pl.ds(16, 16)]
                b0 = b0 + rows_v[base + s + 1, pl.ds(0, 16)]
                b1 = b1 + rows_v[base + s + 1, pl.ds(16, 16)]
            out_v[c, pl.ds(0, 16)] = (a0 + b0) * INV_S
            out_v[c, pl.ds(16, 16)] = (a1 + b1) * INV_S
            return carry2

        lax.fori_loop(0, CHUNK, row_body, 0)
        pltpu.sync_copy(out_v, out_hbm.at[pl.ds(chunk_id * CHUNK, CHUNK)])
        return carry

    lax.fori_loop(0, N_CHUNKS, chunk_body, 0)


def kernel(input, word_vectors):
    idx = input.astype(jnp.int32).reshape(BATCH * SEQ // GATHER, GATHER)
    return _emb(idx, word_vectors)


# speedup vs baseline: 2.7962x; 2.7962x over previous
import functools

import jax
import jax.numpy as jnp
from jax import lax
from jax.experimental import pallas as pl
from jax.experimental.pallas import tpu as pltpu
from jax.experimental.pallas import tpu_sc as plsc

EMBED = 32
BATCH = 16384
SEQ = 50

NC = 2
NS = 16
NW = NC * NS
ROWS_PER_W = BATCH // NW
CHUNK = 64
N_CHUNKS = ROWS_PER_W // CHUNK
IDX_PER_CHUNK = CHUNK * SEQ
GATHER = 128
N_GATHERS = IDX_PER_CHUNK // GATHER
IDX_ROWS_PER_CHUNK = IDX_PER_CHUNK // GATHER
INV_S = 1.0 / SEQ

_mesh = plsc.VectorSubcoreMesh(core_axis_name="c", subcore_axis_name="s")


@functools.partial(
    pl.kernel,
    mesh=_mesh,
    out_type=jax.ShapeDtypeStruct((BATCH, EMBED), jnp.float32),
    compiler_params=pltpu.CompilerParams(use_tc_tiling_on_sc=False),
    scratch_types=[
        pltpu.VMEM((IDX_PER_CHUNK,), jnp.int32),
        pltpu.VMEM((IDX_PER_CHUNK, EMBED), jnp.float32),
        pltpu.VMEM((CHUNK, EMBED), jnp.float32),
        pltpu.SemaphoreType.DMA,
    ],
)
def _emb(idx_hbm, table_hbm, out_hbm, idx_v, rows_v, out_v, sem):
    wid = lax.axis_index("s") * NC + lax.axis_index("c")

    def chunk_body(k, carry):
        chunk_id = wid * N_CHUNKS + k
        pltpu.sync_copy(
            idx_hbm.at[pl.ds(chunk_id * IDX_PER_CHUNK, IDX_PER_CHUNK)],
            idx_v,
        )
        copies = [
            pltpu.async_copy(
                table_hbm.at[idx_v.at[pl.ds(j * GATHER, GATHER)]],
                rows_v.at[pl.ds(j * GATHER, GATHER)],
                sem,
            )
            for j in range(N_GATHERS)
        ]
        for c in copies:
            c.wait()

        def row_body(c, carry2):
            base = c * SEQ
            a0 = rows_v[base, pl.ds(0, 16)]
            a1 = rows_v[base, pl.ds(16, 16)]
            b0 = rows_v[base + 1, pl.ds(0, 16)]
            b1 = rows_v[base + 1, pl.ds(16, 16)]
            for s in range(2, SEQ, 2):
                a0 = a0 + rows_v[base + s, pl.ds(0, 16)]
                a1 = a1 + rows_v[base + s, pl.ds(16, 16)]
                b0 = b0 + rows_v[base + s + 1, pl.ds(0, 16)]
                b1 = b1 + rows_v[base + s + 1, pl.ds(16, 16)]
            out_v[c, pl.ds(0, 16)] = (a0 + b0) * INV_S
            out_v[c, pl.ds(16, 16)] = (a1 + b1) * INV_S
            return carry2

        lax.fori_loop(0, CHUNK, row_body, 0)
        pltpu.sync_copy(out_v, out_hbm.at[pl.ds(chunk_id * CHUNK, CHUNK)])
        return carry

    lax.fori_loop(0, N_CHUNKS, chunk_body, 0)


def kernel(input, word_vectors):
    idx = input.astype(jnp.int32).reshape(BATCH * SEQ)
    return _emb(idx, word_vectors)
